# baseline (speedup 1.0000x reference)
import functools

import jax
import jax.numpy as jnp
from jax import lax
from jax.experimental import pallas as pl
from jax.experimental.pallas import tpu as pltpu
from jax.experimental.pallas import tpu_sc as plsc

N = 10000
E = 320000
IN_CH = 128
HID = 64
OUT = 2
G = 64

NC = 2
NS = 16
NW = NC * NS

CHUNK = 128
NCHUNK = 79
C0 = 79
C1 = 2 * NCHUNK - C0
CMAX = max(C0, C1)
TOT_CHUNKS = NS * (C0 + C1)
E_PAD = TOT_CHUNKS * CHUNK

RPT = 632
N_ACC = RPT * NS
DUMMY = N + 8

BN = 1000
NBUF = 2

_PREC = lax.Precision.HIGHEST


def _sc_mesh():
    return plsc.VectorSubcoreMesh(
        core_axis_name="c", subcore_axis_name="s", num_cores=NC, num_subcores=NS
    )


def _deg_body(dstp_hbm, ones8_hbm, zero8_hbm, out_hbm, dst_v, ones_v, wout_v,
              acc_sh):
    cid = lax.axis_index("c")
    sid = lax.axis_index("s")
    row0 = sid * RPT

    pltpu.sync_copy(ones8_hbm, ones_v)
    pltpu.sync_copy(zero8_hbm, wout_v)
    pltpu.sync_copy(wout_v, acc_sh.at[pl.ds(row0, RPT)])
    plsc.subcore_barrier()

    def run(m, base):
        pltpu.sync_copy(dstp_hbm.at[pl.ds(base, m)], dst_v.at[pl.ds(0, m)])

        def body(j, _):
            pltpu.sync_copy(ones_v, acc_sh.at[dst_v.at[j]], add=True)
            return ()

        lax.fori_loop(0, m, body, (), unroll=False)

    @pl.when(cid == 0)
    def _():
        run(C0, sid * C0)

    @pl.when(cid == 1)
    def _():
        run(C1, NS * C0 + sid * C1)

    plsc.subcore_barrier()
    pltpu.sync_copy(acc_sh.at[pl.ds(row0, RPT)], wout_v)
    pltpu.sync_copy(wout_v, out_hbm.at[cid, pl.ds(row0, RPT)])


_deg_call = pl.kernel(
    _deg_body,
    out_type=jax.ShapeDtypeStruct((NC, N_ACC, 8), jnp.float32),
    mesh=_sc_mesh(),
    compiler_params=pltpu.CompilerParams(use_tc_tiling_on_sc=False),
    scratch_types=[
        pltpu.VMEM((CMAX, CHUNK), jnp.int32),
        pltpu.VMEM((CHUNK, 8), jnp.float32),
        pltpu.VMEM((RPT, 8), jnp.float32),
        pltpu.VMEM_SHARED((N_ACC, 8), jnp.float32),
    ],
)


def _agg_body(hs_hbm, srcp_hbm, dstp_hbm, zero64_hbm, out_hbm,
              src_v, dst_v, rows_v, wout_v, acc_sh, gsem, ssem):
    cid = lax.axis_index("c")
    sid = lax.axis_index("s")
    row0 = sid * RPT

    pltpu.sync_copy(zero64_hbm, rows_v.at[0])
    for k in range(4):
        pltpu.sync_copy(rows_v.at[0], acc_sh.at[pl.ds(row0 + k * CHUNK, CHUNK)])
    pltpu.sync_copy(rows_v.at[0, pl.ds(0, RPT - 4 * CHUNK)],
                    acc_sh.at[pl.ds(row0 + 4 * CHUNK, RPT - 4 * CHUNK)])
    plsc.subcore_barrier()

    def run(m, base):
        pltpu.sync_copy(srcp_hbm.at[pl.ds(base, m)], src_v.at[pl.ds(0, m)])
        pltpu.sync_copy(dstp_hbm.at[pl.ds(base, m)], dst_v.at[pl.ds(0, m)])

        def start_g(j, b):
            pltpu.async_copy(hs_hbm.at[src_v.at[j]], rows_v.at[b], gsem.at[b])

        def wait_g(j, b):
            pltpu.make_async_copy(hs_hbm.at[src_v.at[j]], rows_v.at[b],
                                  gsem.at[b]).wait()

        for j in range(NBUF - 1):
            start_g(j, j)

        def body(j, _):
            b = lax.rem(j, NBUF)
            wait_g(j, b)
            pltpu.sync_copy(rows_v.at[b], acc_sh.at[dst_v.at[j]], add=True)
            jn = j + NBUF - 1

            @pl.when(jn < m)
            def _():
                start_g(jn, lax.rem(jn, NBUF))

            return ()

        lax.fori_loop(0, m, body, (), unroll=False)

    @pl.when(cid == 0)
    def _():
        run(C0, sid * C0)

    @pl.when(cid == 1)
    def _():
        run(C1, NS * C0 + sid * C1)

    plsc.subcore_barrier()
    pltpu.sync_copy(acc_sh.at[pl.ds(row0, RPT)], wout_v)
    pltpu.sync_copy(wout_v, out_hbm.at[cid, pl.ds(row0, RPT)])


_agg_call = pl.kernel(
    _agg_body,
    out_type=jax.ShapeDtypeStruct((NC, N_ACC, HID), jnp.float32),
    mesh=_sc_mesh(),
    compiler_params=pltpu.CompilerParams(use_tc_tiling_on_sc=False),
    scratch_types=[
        pltpu.VMEM((CMAX, CHUNK), jnp.int32),
        pltpu.VMEM((CMAX, CHUNK), jnp.int32),
        pltpu.VMEM((NBUF, CHUNK, HID), jnp.float32),
        pltpu.VMEM((RPT, HID), jnp.float32),
        pltpu.VMEM_SHARED((N_ACC, HID), jnp.float32),
        pltpu.SemaphoreType.DMA((NBUF,)),
        pltpu.SemaphoreType.DMA((2,)),
    ],
)


def _tc1_body(x_ref, w_ref, c_ref, hs_ref, dinv_ref):
    s = c_ref[0, :, 0:1] + c_ref[1, :, 0:1]
    dinv = lax.rsqrt(1.0 + s)
    h = jnp.dot(x_ref[...], w_ref[...],
                preferred_element_type=jnp.float32)
    hs_ref[...] = dinv * h
    dinv_ref[...] = dinv


_tc1_call = pl.pallas_call(
    _tc1_body,
    grid=(N // BN,),
    in_specs=[
        pl.BlockSpec((BN, IN_CH), lambda i: (i, 0)),
        pl.BlockSpec((IN_CH, HID), lambda i: (0, 0)),
        pl.BlockSpec((NC, BN, 8), lambda i: (0, i, 0)),
    ],
    out_specs=[
        pl.BlockSpec((BN, HID), lambda i: (i, 0)),
        pl.BlockSpec((BN, 1), lambda i: (i, 0)),
    ],
    out_shape=[
        jax.ShapeDtypeStruct((N, HID), jnp.float32),
        jax.ShapeDtypeStruct((N, 1), jnp.float32),
    ],
)


def _tc2_body(p_ref, hs1_ref, dinv_ref, b_ref, w_ref, hs2_ref):
    dinv = dinv_ref[...]
    h = p_ref[0] + p_ref[1] + hs1_ref[...]
    h = jnp.maximum(dinv * h + b_ref[...], 0.0)
    hs2_ref[...] = dinv * jnp.dot(h, w_ref[...],
                                  preferred_element_type=jnp.float32)


_tc2_call = pl.pallas_call(
    _tc2_body,
    grid=(N // BN,),
    in_specs=[
        pl.BlockSpec((NC, BN, HID), lambda i: (0, i, 0)),
        pl.BlockSpec((BN, HID), lambda i: (i, 0)),
        pl.BlockSpec((BN, 1), lambda i: (i, 0)),
        pl.BlockSpec((1, HID), lambda i: (0, 0)),
        pl.BlockSpec((HID, HID), lambda i: (0, 0)),
    ],
    out_specs=pl.BlockSpec((BN, HID), lambda i: (i, 0)),
    out_shape=jax.ShapeDtypeStruct((N, HID), jnp.float32),
)


def _tc3_body(p_ref, hs2_ref, dinv_ref, b_ref, batch_ref, wc_ref, bc_ref,
              out_ref, seg_acc, cnt_acc):
    i = pl.program_id(0)

    @pl.when(i == 0)
    def _():
        seg_acc[...] = jnp.zeros_like(seg_acc)
        cnt_acc[...] = jnp.zeros_like(cnt_acc)

    dinv = dinv_ref[...]
    h = p_ref[0] + p_ref[1] + hs2_ref[...]
    h = jnp.maximum(dinv * h + b_ref[...], 0.0)

    gid = lax.broadcasted_iota(jnp.int32, (BN, G), 1)
    onehot = (batch_ref[...] == gid).astype(jnp.float32)
    seg_acc[...] += lax.dot_general(onehot, h, (((0,), (0,)), ((), ())),
                                    precision=_PREC,
                                    preferred_element_type=jnp.float32)
    cnt_acc[...] += lax.dot_general(onehot, jnp.ones((BN, 1), jnp.float32),
                                    (((0,), (0,)), ((), ())),
                                    precision=_PREC,
                                    preferred_element_type=jnp.float32)

    @pl.when(i == pl.num_programs(0) - 1)
    def _():
        mean = seg_acc[...] / jnp.maximum(cnt_acc[...], 1.0)
        out_ref[...] = jnp.dot(mean, wc_ref[...],
                               preferred_element_type=jnp.float32) + bc_ref[...]


_tc3_call = pl.pallas_call(
    _tc3_body,
    grid=(N // BN,),
    in_specs=[
        pl.BlockSpec((NC, BN, HID), lambda i: (0, i, 0)),
        pl.BlockSpec((BN, HID), lambda i: (i, 0)),
        pl.BlockSpec((BN, 1), lambda i: (i, 0)),
        pl.BlockSpec((1, HID), lambda i: (0, 0)),
        pl.BlockSpec((BN, 1), lambda i: (i, 0)),
        pl.BlockSpec((HID, OUT), lambda i: (0, 0)),
        pl.BlockSpec((1, OUT), lambda i: (0, 0)),
    ],
    out_specs=pl.BlockSpec((G, OUT), lambda i: (0, 0)),
    out_shape=jax.ShapeDtypeStruct((G, OUT), jnp.float32),
    scratch_shapes=[
        pltpu.VMEM((G, HID), jnp.float32),
        pltpu.VMEM((G, 1), jnp.float32),
    ],
)


def kernel(x, edge_index, batch, W1, b1, W2, b2, Wc, bc):
    src = edge_index[0].astype(jnp.int32)
    dst = edge_index[1].astype(jnp.int32)
    pad = E_PAD - E
    srcp = jnp.concatenate([src, jnp.zeros((pad,), jnp.int32)]).reshape(TOT_CHUNKS, CHUNK)
    dstp = jnp.concatenate([dst, jnp.full((pad,), DUMMY, jnp.int32)]).reshape(TOT_CHUNKS, CHUNK)

    ones8 = jnp.zeros((CHUNK, 8), jnp.float32).at[:, 0].set(1.0)
    zero8 = jnp.zeros((RPT, 8), jnp.float32)
    zero64 = jnp.zeros((CHUNK, HID), jnp.float32)

    c = _deg_call(dstp, ones8, zero8)
    hs1, dinv = _tc1_call(x, W1, c[:, :N, :])

    p1 = _agg_call(hs1, srcp, dstp, zero64)
    hs2 = _tc2_call(p1[:, :N, :], hs1, dinv, b1.reshape(1, HID), W2)

    p2 = _agg_call(hs2, srcp, dstp, zero64)
    out = _tc3_call(p2[:, :N, :], hs2, dinv, b2.reshape(1, HID),
                    batch.astype(jnp.int32).reshape(N, 1), Wc, bc.reshape(1, OUT))
    return out


# speedup vs baseline: 1.1113x; 1.1113x over previous
import functools

import jax
import jax.numpy as jnp
from jax import lax
from jax.experimental import pallas as pl
from jax.experimental.pallas import tpu as pltpu
from jax.experimental.pallas import tpu_sc as plsc

N = 10000
E = 320000
IN_CH = 128
HID = 64
OUT = 2
G = 64

NC = 2
NS = 16
NW = NC * NS

CHUNK = 128
NCHUNK = 79
C0 = 79
C1 = 2 * NCHUNK - C0
CMAX = max(C0, C1)
TOT_CHUNKS = NS * (C0 + C1)
E_PAD = TOT_CHUNKS * CHUNK

RPT = 632
N_ACC = RPT * NS
DUMMY = N + 8

BN = 1000
NBUF = 2

_PREC = lax.Precision.HIGHEST


def _sc_mesh():
    return plsc.VectorSubcoreMesh(
        core_axis_name="c", subcore_axis_name="s", num_cores=NC, num_subcores=NS
    )


def _deg_body(dstp_hbm, ones8_hbm, zero8_hbm, out_hbm, dst_v, ones_v, wout_v,
              acc_sh):
    cid = lax.axis_index("c")
    sid = lax.axis_index("s")
    row0 = sid * RPT

    pltpu.sync_copy(ones8_hbm, ones_v)
    pltpu.sync_copy(zero8_hbm, wout_v)
    pltpu.sync_copy(wout_v, acc_sh.at[pl.ds(row0, RPT)])
    plsc.subcore_barrier()

    def run(m, base):
        pltpu.sync_copy(dstp_hbm.at[pl.ds(base, m)], dst_v.at[pl.ds(0, m)])

        def body(j, _):
            pltpu.sync_copy(ones_v, acc_sh.at[dst_v.at[j]], add=True)
            return ()

        lax.fori_loop(0, m, body, (), unroll=False)

    @pl.when(cid == 0)
    def _():
        run(C0, sid * C0)

    @pl.when(cid == 1)
    def _():
        run(C1, NS * C0 + sid * C1)

    plsc.subcore_barrier()
    pltpu.sync_copy(acc_sh.at[pl.ds(row0, RPT)], wout_v)
    pltpu.sync_copy(wout_v, out_hbm.at[cid, pl.ds(row0, RPT)])


_deg_call = pl.kernel(
    _deg_body,
    out_type=jax.ShapeDtypeStruct((NC, N_ACC, 8), jnp.float32),
    mesh=_sc_mesh(),
    compiler_params=pltpu.CompilerParams(use_tc_tiling_on_sc=False),
    scratch_types=[
        pltpu.VMEM((CMAX, CHUNK), jnp.int32),
        pltpu.VMEM((CHUNK, 8), jnp.float32),
        pltpu.VMEM((RPT, 8), jnp.float32),
        pltpu.VMEM_SHARED((N_ACC, 8), jnp.float32),
    ],
)


def _agg_body(hs_hbm, srcp_hbm, dstp_hbm, zero64_hbm, out_hbm,
              src_v, dst_v, rows_v, wout_v, acc_sh, gsem, ssem):
    cid = lax.axis_index("c")
    sid = lax.axis_index("s")
    row0 = sid * RPT

    pltpu.sync_copy(zero64_hbm, rows_v.at[0])
    for k in range(4):
        pltpu.sync_copy(rows_v.at[0], acc_sh.at[pl.ds(row0 + k * CHUNK, CHUNK)])
    pltpu.sync_copy(rows_v.at[0, pl.ds(0, RPT - 4 * CHUNK)],
                    acc_sh.at[pl.ds(row0 + 4 * CHUNK, RPT - 4 * CHUNK)])
    plsc.subcore_barrier()

    def run(m, base):
        pltpu.sync_copy(srcp_hbm.at[pl.ds(base, m)], src_v.at[pl.ds(0, m)])
        pltpu.sync_copy(dstp_hbm.at[pl.ds(base, m)], dst_v.at[pl.ds(0, m)])

        def start_g(j, b):
            pltpu.async_copy(hs_hbm.at[src_v.at[j]], rows_v.at[b], gsem.at[b])

        def wait_g(j, b):
            pltpu.make_async_copy(hs_hbm.at[src_v.at[j]], rows_v.at[b],
                                  gsem.at[b]).wait()

        for j in range(NBUF - 1):
            start_g(j, j)

        def body(j, _):
            b = lax.rem(j, NBUF)
            wait_g(j, b)

            @pl.when(j + 1 < m)
            def _():
                start_g(j + 1, 1 - b)

            pltpu.sync_copy(rows_v.at[b], acc_sh.at[dst_v.at[j]], add=True)
            return ()

        lax.fori_loop(0, m, body, (), unroll=False)

    @pl.when(cid == 0)
    def _():
        run(C0, sid * C0)

    @pl.when(cid == 1)
    def _():
        run(C1, NS * C0 + sid * C1)

    plsc.subcore_barrier()
    pltpu.sync_copy(acc_sh.at[pl.ds(row0, RPT)], wout_v)
    pltpu.sync_copy(wout_v, out_hbm.at[cid, pl.ds(row0, RPT)])


_agg_call = pl.kernel(
    _agg_body,
    out_type=jax.ShapeDtypeStruct((NC, N_ACC, HID), jnp.float32),
    mesh=_sc_mesh(),
    compiler_params=pltpu.CompilerParams(use_tc_tiling_on_sc=False),
    scratch_types=[
        pltpu.VMEM((CMAX, CHUNK), jnp.int32),
        pltpu.VMEM((CMAX, CHUNK), jnp.int32),
        pltpu.VMEM((NBUF, CHUNK, HID), jnp.float32),
        pltpu.VMEM((RPT, HID), jnp.float32),
        pltpu.VMEM_SHARED((N_ACC, HID), jnp.float32),
        pltpu.SemaphoreType.DMA((NBUF,)),
        pltpu.SemaphoreType.DMA((2,)),
    ],
)


def _tc1_body(x_ref, w_ref, c_ref, hs_ref, dinv_ref):
    s = c_ref[0, :, 0:1] + c_ref[1, :, 0:1]
    dinv = lax.rsqrt(1.0 + s)
    h = jnp.dot(x_ref[...], w_ref[...],
                preferred_element_type=jnp.float32)
    hs_ref[...] = dinv * h
    dinv_ref[...] = dinv


_tc1_call = pl.pallas_call(
    _tc1_body,
    grid=(N // BN,),
    in_specs=[
        pl.BlockSpec((BN, IN_CH), lambda i: (i, 0)),
        pl.BlockSpec((IN_CH, HID), lambda i: (0, 0)),
        pl.BlockSpec((NC, BN, 8), lambda i: (0, i, 0)),
    ],
    out_specs=[
        pl.BlockSpec((BN, HID), lambda i: (i, 0)),
        pl.BlockSpec((BN, 1), lambda i: (i, 0)),
    ],
    out_shape=[
        jax.ShapeDtypeStruct((N, HID), jnp.float32),
        jax.ShapeDtypeStruct((N, 1), jnp.float32),
    ],
)


def _tc2_body(p_ref, hs1_ref, dinv_ref, b_ref, w_ref, hs2_ref):
    dinv = dinv_ref[...]
    h = p_ref[0] + p_ref[1] + hs1_ref[...]
    h = jnp.maximum(dinv * h + b_ref[...], 0.0)
    hs2_ref[...] = dinv * jnp.dot(h, w_ref[...],
                                  preferred_element_type=jnp.float32)


_tc2_call = pl.pallas_call(
    _tc2_body,
    grid=(N // BN,),
    in_specs=[
        pl.BlockSpec((NC, BN, HID), lambda i: (0, i, 0)),
        pl.BlockSpec((BN, HID), lambda i: (i, 0)),
        pl.BlockSpec((BN, 1), lambda i: (i, 0)),
        pl.BlockSpec((1, HID), lambda i: (0, 0)),
        pl.BlockSpec((HID, HID), lambda i: (0, 0)),
    ],
    out_specs=pl.BlockSpec((BN, HID), lambda i: (i, 0)),
    out_shape=jax.ShapeDtypeStruct((N, HID), jnp.float32),
)


def _tc3_body(p_ref, hs2_ref, dinv_ref, b_ref, batch_ref, wc_ref, bc_ref,
              out_ref, seg_acc, cnt_acc):
    i = pl.program_id(0)

    @pl.when(i == 0)
    def _():
        seg_acc[...] = jnp.zeros_like(seg_acc)
        cnt_acc[...] = jnp.zeros_like(cnt_acc)

    dinv = dinv_ref[...]
    h = p_ref[0] + p_ref[1] + hs2_ref[...]
    h = jnp.maximum(dinv * h + b_ref[...], 0.0)

    gid = lax.broadcasted_iota(jnp.int32, (BN, G), 1)
    onehot = (batch_ref[...] == gid).astype(jnp.float32)
    seg_acc[...] += lax.dot_general(onehot, h, (((0,), (0,)), ((), ())),
                                    precision=_PREC,
                                    preferred_element_type=jnp.float32)
    cnt_acc[...] += lax.dot_general(onehot, jnp.ones((BN, 1), jnp.float32),
                                    (((0,), (0,)), ((), ())),
                                    precision=_PREC,
                                    preferred_element_type=jnp.float32)

    @pl.when(i == pl.num_programs(0) - 1)
    def _():
        mean = seg_acc[...] / jnp.maximum(cnt_acc[...], 1.0)
        out_ref[...] = jnp.dot(mean, wc_ref[...],
                               preferred_element_type=jnp.float32) + bc_ref[...]


_tc3_call = pl.pallas_call(
    _tc3_body,
    grid=(N // BN,),
    in_specs=[
        pl.BlockSpec((NC, BN, HID), lambda i: (0, i, 0)),
        pl.BlockSpec((BN, HID), lambda i: (i, 0)),
        pl.BlockSpec((BN, 1), lambda i: (i, 0)),
        pl.BlockSpec((1, HID), lambda i: (0, 0)),
        pl.BlockSpec((BN, 1), lambda i: (i, 0)),
        pl.BlockSpec((HID, OUT), lambda i: (0, 0)),
        pl.BlockSpec((1, OUT), lambda i: (0, 0)),
    ],
    out_specs=pl.BlockSpec((G, OUT), lambda i: (0, 0)),
    out_shape=jax.ShapeDtypeStruct((G, OUT), jnp.float32),
    scratch_shapes=[
        pltpu.VMEM((G, HID), jnp.float32),
        pltpu.VMEM((G, 1), jnp.float32),
    ],
)


def kernel(x, edge_index, batch, W1, b1, W2, b2, Wc, bc):
    src = edge_index[0].astype(jnp.int32)
    dst = edge_index[1].astype(jnp.int32)
    pad = E_PAD - E
    srcp = jnp.concatenate([src, jnp.zeros((pad,), jnp.int32)]).reshape(TOT_CHUNKS, CHUNK)
    dstp = jnp.concatenate([dst, jnp.full((pad,), DUMMY, jnp.int32)]).reshape(TOT_CHUNKS, CHUNK)

    ones8 = jnp.zeros((CHUNK, 8), jnp.float32).at[:, 0].set(1.0)
    zero8 = jnp.zeros((RPT, 8), jnp.float32)
    zero64 = jnp.zeros((CHUNK, HID), jnp.float32)

    c = _deg_call(dstp, ones8, zero8)
    hs1, dinv = _tc1_call(x, W1, c[:, :N, :])

    p1 = _agg_call(hs1, srcp, dstp, zero64)
    hs2 = _tc2_call(p1[:, :N, :], hs1, dinv, b1.reshape(1, HID), W2)

    p2 = _agg_call(hs2, srcp, dstp, zero64)
    out = _tc3_call(p2[:, :N, :], hs2, dinv, b2.reshape(1, HID),
                    batch.astype(jnp.int32).reshape(N, 1), Wc, bc.reshape(1, OUT))
    return out


# speedup vs baseline: 1.2111x; 1.0899x over previous
import functools

import jax
import jax.numpy as jnp
from jax import lax
from jax.experimental import pallas as pl
from jax.experimental.pallas import tpu as pltpu
from jax.experimental.pallas import tpu_sc as plsc

N = 10000
E = 320000
IN_CH = 128
HID = 64
OUT = 2
G = 64

NC = 2
NS = 16
NW = NC * NS

CHUNK = 128
NCHUNK = 79
C0 = 102
C1 = 2 * NCHUNK - C0
CMAX = max(C0, C1)
TOT_CHUNKS = NS * (C0 + C1)
E_PAD = TOT_CHUNKS * CHUNK

RPT = 632
N_ACC = RPT * NS
DUMMY = N + 8

BN = 1000
NBUF = 2

_PREC = lax.Precision.HIGHEST


def _sc_mesh():
    return plsc.VectorSubcoreMesh(
        core_axis_name="c", subcore_axis_name="s", num_cores=NC, num_subcores=NS
    )


def _deg_body(dstp_hbm, ones8_hbm, zero8_hbm, out_hbm, dst_v, ones_v, wout_v,
              acc_sh):
    cid = lax.axis_index("c")
    sid = lax.axis_index("s")
    row0 = sid * RPT

    pltpu.sync_copy(ones8_hbm, ones_v)
    pltpu.sync_copy(zero8_hbm, wout_v)
    pltpu.sync_copy(wout_v, acc_sh.at[pl.ds(row0, RPT)])
    plsc.subcore_barrier()

    def run(m, base):
        pltpu.sync_copy(dstp_hbm.at[pl.ds(base, m)], dst_v.at[pl.ds(0, m)])

        def body(j, _):
            pltpu.sync_copy(ones_v, acc_sh.at[dst_v.at[j]], add=True)
            return ()

        lax.fori_loop(0, m, body, (), unroll=False)

    @pl.when(cid == 0)
    def _():
        run(C0, sid * C0)

    @pl.when(cid == 1)
    def _():
        run(C1, NS * C0 + sid * C1)

    plsc.subcore_barrier()
    pltpu.sync_copy(acc_sh.at[pl.ds(row0, RPT)], wout_v)
    pltpu.sync_copy(wout_v, out_hbm.at[cid, pl.ds(row0, RPT)])


_deg_call = pl.kernel(
    _deg_body,
    out_type=jax.ShapeDtypeStruct((NC, N_ACC, 8), jnp.float32),
    mesh=_sc_mesh(),
    compiler_params=pltpu.CompilerParams(use_tc_tiling_on_sc=False),
    scratch_types=[
        pltpu.VMEM((CMAX, CHUNK), jnp.int32),
        pltpu.VMEM((CHUNK, 8), jnp.float32),
        pltpu.VMEM((RPT, 8), jnp.float32),
        pltpu.VMEM_SHARED((N_ACC, 8), jnp.float32),
    ],
)


def _agg_body(hs_hbm, srcp_hbm, dstp_hbm, zero64_hbm, out_hbm,
              src_v, dst_v, rows_v, wout_v, acc_sh, gsem, ssem):
    cid = lax.axis_index("c")
    sid = lax.axis_index("s")
    row0 = sid * RPT

    pltpu.sync_copy(zero64_hbm, rows_v.at[0])
    for k in range(4):
        pltpu.sync_copy(rows_v.at[0], acc_sh.at[pl.ds(row0 + k * CHUNK, CHUNK)])
    pltpu.sync_copy(rows_v.at[0, pl.ds(0, RPT - 4 * CHUNK)],
                    acc_sh.at[pl.ds(row0 + 4 * CHUNK, RPT - 4 * CHUNK)])
    plsc.subcore_barrier()

    def run(m, base):
        pltpu.sync_copy(srcp_hbm.at[pl.ds(base, m)], src_v.at[pl.ds(0, m)])
        pltpu.sync_copy(dstp_hbm.at[pl.ds(base, m)], dst_v.at[pl.ds(0, m)])

        def start_g(j, b):
            pltpu.async_copy(hs_hbm.at[src_v.at[j]], rows_v.at[b], gsem.at[b])

        def wait_g(j, b):
            pltpu.make_async_copy(hs_hbm.at[src_v.at[j]], rows_v.at[b],
                                  gsem.at[b]).wait()

        for j in range(NBUF - 1):
            start_g(j, j)

        def body(j, _):
            b = lax.rem(j, NBUF)
            wait_g(j, b)

            @pl.when(j + 1 < m)
            def _():
                start_g(j + 1, 1 - b)

            pltpu.sync_copy(rows_v.at[b], acc_sh.at[dst_v.at[j]], add=True)
            return ()

        lax.fori_loop(0, m, body, (), unroll=False)

    @pl.when(cid == 0)
    def _():
        run(C0, sid * C0)

    @pl.when(cid == 1)
    def _():
        run(C1, NS * C0 + sid * C1)

    plsc.subcore_barrier()
    pltpu.sync_copy(acc_sh.at[pl.ds(row0, RPT)], wout_v)
    pltpu.sync_copy(wout_v, out_hbm.at[cid, pl.ds(row0, RPT)])


_agg_call = pl.kernel(
    _agg_body,
    out_type=jax.ShapeDtypeStruct((NC, N_ACC, HID), jnp.float32),
    mesh=_sc_mesh(),
    compiler_params=pltpu.CompilerParams(use_tc_tiling_on_sc=False),
    scratch_types=[
        pltpu.VMEM((CMAX, CHUNK), jnp.int32),
        pltpu.VMEM((CMAX, CHUNK), jnp.int32),
        pltpu.VMEM((NBUF, CHUNK, HID), jnp.float32),
        pltpu.VMEM((RPT, HID), jnp.float32),
        pltpu.VMEM_SHARED((N_ACC, HID), jnp.float32),
        pltpu.SemaphoreType.DMA((NBUF,)),
        pltpu.SemaphoreType.DMA((2,)),
    ],
)


def _tc1_body(x_ref, w_ref, c_ref, hs_ref, dinv_ref):
    s = c_ref[0, :, 0:1] + c_ref[1, :, 0:1]
    dinv = lax.rsqrt(1.0 + s)
    h = jnp.dot(x_ref[...], w_ref[...],
                preferred_element_type=jnp.float32)
    hs_ref[...] = dinv * h
    dinv_ref[...] = dinv


_tc1_call = pl.pallas_call(
    _tc1_body,
    grid=(N // BN,),
    in_specs=[
        pl.BlockSpec((BN, IN_CH), lambda i: (i, 0)),
        pl.BlockSpec((IN_CH, HID), lambda i: (0, 0)),
        pl.BlockSpec((NC, BN, 8), lambda i: (0, i, 0)),
    ],
    out_specs=[
        pl.BlockSpec((BN, HID), lambda i: (i, 0)),
        pl.BlockSpec((BN, 1), lambda i: (i, 0)),
    ],
    out_shape=[
        jax.ShapeDtypeStruct((N, HID), jnp.float32),
        jax.ShapeDtypeStruct((N, 1), jnp.float32),
    ],
)


def _tc2_body(p_ref, hs1_ref, dinv_ref, b_ref, w_ref, hs2_ref):
    dinv = dinv_ref[...]
    h = p_ref[0] + p_ref[1] + hs1_ref[...]
    h = jnp.maximum(dinv * h + b_ref[...], 0.0)
    hs2_ref[...] = dinv * jnp.dot(h, w_ref[...],
                                  preferred_element_type=jnp.float32)


_tc2_call = pl.pallas_call(
    _tc2_body,
    grid=(N // BN,),
    in_specs=[
        pl.BlockSpec((NC, BN, HID), lambda i: (0, i, 0)),
        pl.BlockSpec((BN, HID), lambda i: (i, 0)),
        pl.BlockSpec((BN, 1), lambda i: (i, 0)),
        pl.BlockSpec((1, HID), lambda i: (0, 0)),
        pl.BlockSpec((HID, HID), lambda i: (0, 0)),
    ],
    out_specs=pl.BlockSpec((BN, HID), lambda i: (i, 0)),
    out_shape=jax.ShapeDtypeStruct((N, HID), jnp.float32),
)


def _tc3_body(p_ref, hs2_ref, dinv_ref, b_ref, batch_ref, wc_ref, bc_ref,
              out_ref, seg_acc, cnt_acc):
    i = pl.program_id(0)

    @pl.when(i == 0)
    def _():
        seg_acc[...] = jnp.zeros_like(seg_acc)
        cnt_acc[...] = jnp.zeros_like(cnt_acc)

    dinv = dinv_ref[...]
    h = p_ref[0] + p_ref[1] + hs2_ref[...]
    h = jnp.maximum(dinv * h + b_ref[...], 0.0)

    gid = lax.broadcasted_iota(jnp.int32, (BN, G), 1)
    onehot = (batch_ref[...] == gid).astype(jnp.float32)
    seg_acc[...] += lax.dot_general(onehot, h, (((0,), (0,)), ((), ())),
                                    precision=_PREC,
                                    preferred_element_type=jnp.float32)
    cnt_acc[...] += lax.dot_general(onehot, jnp.ones((BN, 1), jnp.float32),
                                    (((0,), (0,)), ((), ())),
                                    precision=_PREC,
                                    preferred_element_type=jnp.float32)

    @pl.when(i == pl.num_programs(0) - 1)
    def _():
        mean = seg_acc[...] / jnp.maximum(cnt_acc[...], 1.0)
        out_ref[...] = jnp.dot(mean, wc_ref[...],
                               preferred_element_type=jnp.float32) + bc_ref[...]


_tc3_call = pl.pallas_call(
    _tc3_body,
    grid=(N // BN,),
    in_specs=[
        pl.BlockSpec((NC, BN, HID), lambda i: (0, i, 0)),
        pl.BlockSpec((BN, HID), lambda i: (i, 0)),
        pl.BlockSpec((BN, 1), lambda i: (i, 0)),
        pl.BlockSpec((1, HID), lambda i: (0, 0)),
        pl.BlockSpec((BN, 1), lambda i: (i, 0)),
        pl.BlockSpec((HID, OUT), lambda i: (0, 0)),
        pl.BlockSpec((1, OUT), lambda i: (0, 0)),
    ],
    out_specs=pl.BlockSpec((G, OUT), lambda i: (0, 0)),
    out_shape=jax.ShapeDtypeStruct((G, OUT), jnp.float32),
    scratch_shapes=[
        pltpu.VMEM((G, HID), jnp.float32),
        pltpu.VMEM((G, 1), jnp.float32),
    ],
)


def kernel(x, edge_index, batch, W1, b1, W2, b2, Wc, bc):
    src = edge_index[0].astype(jnp.int32)
    dst = edge_index[1].astype(jnp.int32)
    pad = E_PAD - E
    srcp = jnp.concatenate([src, jnp.zeros((pad,), jnp.int32)]).reshape(TOT_CHUNKS, CHUNK)
    dstp = jnp.concatenate([dst, jnp.full((pad,), DUMMY, jnp.int32)]).reshape(TOT_CHUNKS, CHUNK)

    ones8 = jnp.zeros((CHUNK, 8), jnp.float32).at[:, 0].set(1.0)
    zero8 = jnp.zeros((RPT, 8), jnp.float32)
    zero64 = jnp.zeros((CHUNK, HID), jnp.float32)

    c = _deg_call(dstp, ones8, zero8)
    hs1, dinv = _tc1_call(x, W1, c[:, :N, :])

    p1 = _agg_call(hs1, srcp, dstp, zero64)
    hs2 = _tc2_call(p1[:, :N, :], hs1, dinv, b1.reshape(1, HID), W2)

    p2 = _agg_call(hs2, srcp, dstp, zero64)
    out = _tc3_call(p2[:, :N, :], hs2, dinv, b2.reshape(1, HID),
                    batch.astype(jnp.int32).reshape(N, 1), Wc, bc.reshape(1, OUT))
    return out


# speedup vs baseline: 1.2140x; 1.0024x over previous
import jax
import jax.numpy as jnp
from jax import lax
from jax.experimental import pallas as pl
from jax.experimental.pallas import tpu as pltpu
from jax.experimental.pallas import tpu_sc as plsc

N = 10000
E = 320000
IN_CH = 128
HID = 64
OUT = 2
G = 64

NC = 2
NS = 16
NW = NC * NS

CHUNK = 128
EPW = E // NW
NCHUNK = -(-EPW // CHUNK)
EPW_PAD = NCHUNK * CHUNK
E_PAD = EPW_PAD * NW

RPT = 632
N_ACC = RPT * NS
DUMMY = N + 8

BN = 1000
NBUF = 3

_PREC = lax.Precision.HIGHEST


def _sc_mesh():
    return plsc.VectorSubcoreMesh(
        core_axis_name="c", subcore_axis_name="s", num_cores=NC, num_subcores=NS
    )


def _deg_body(dstp_hbm, ones8_hbm, zero8_hbm, out_hbm, dst_v, ones_v, wout_v,
              acc_sh):
    cid = lax.axis_index("c")
    sid = lax.axis_index("s")
    wid = sid * NC + cid
    row0 = sid * RPT

    pltpu.sync_copy(dstp_hbm.at[wid], dst_v)
    pltpu.sync_copy(ones8_hbm, ones_v)
    pltpu.sync_copy(zero8_hbm, wout_v)
    pltpu.sync_copy(wout_v, acc_sh.at[pl.ds(row0, RPT)])
    plsc.subcore_barrier()

    def body(j, _):
        pltpu.sync_copy(ones_v, acc_sh.at[dst_v.at[j]], add=True)
        return ()

    lax.fori_loop(0, NCHUNK, body, (), unroll=False)

    plsc.subcore_barrier()
    pltpu.sync_copy(acc_sh.at[pl.ds(row0, RPT)], wout_v)
    pltpu.sync_copy(wout_v, out_hbm.at[cid, pl.ds(row0, RPT)])


_deg_call = pl.kernel(
    _deg_body,
    out_type=jax.ShapeDtypeStruct((NC, N_ACC, 8), jnp.float32),
    mesh=_sc_mesh(),
    compiler_params=pltpu.CompilerParams(use_tc_tiling_on_sc=False),
    scratch_types=[
        pltpu.VMEM((NCHUNK, CHUNK), jnp.int32),
        pltpu.VMEM((CHUNK, 8), jnp.float32),
        pltpu.VMEM((RPT, 8), jnp.float32),
        pltpu.VMEM_SHARED((N_ACC, 8), jnp.float32),
    ],
)


def _agg_body(hs_hbm, srcp_hbm, dstp_hbm, zero64_hbm, out_hbm,
              src_v, dst_v, rows_v, wout_v, acc_sh, gsem):
    cid = lax.axis_index("c")
    sid = lax.axis_index("s")
    wid = sid * NC + cid
    row0 = sid * RPT

    pltpu.sync_copy(srcp_hbm.at[wid], src_v)
    pltpu.sync_copy(dstp_hbm.at[wid], dst_v)

    pltpu.sync_copy(zero64_hbm, rows_v.at[0])
    for k in range(4):
        pltpu.sync_copy(rows_v.at[0], acc_sh.at[pl.ds(row0 + k * CHUNK, CHUNK)])
    pltpu.sync_copy(rows_v.at[0, pl.ds(0, RPT - 4 * CHUNK)],
                    acc_sh.at[pl.ds(row0 + 4 * CHUNK, RPT - 4 * CHUNK)])
    plsc.subcore_barrier()

    def start_g(j, b):
        pltpu.async_copy(hs_hbm.at[src_v.at[j]], rows_v.at[b], gsem.at[b])

    def wait_g(j, b):
        pltpu.make_async_copy(hs_hbm.at[src_v.at[j]], rows_v.at[b], gsem.at[b]).wait()

    for j in range(NBUF - 1):
        start_g(j, j)

    def body(j, _):
        b = lax.rem(j, NBUF)
        wait_g(j, b)
        pltpu.sync_copy(rows_v.at[b], acc_sh.at[dst_v.at[j]], add=True)
        jn = j + NBUF - 1

        @pl.when(jn < NCHUNK)
        def _():
            start_g(jn, lax.rem(jn, NBUF))

        return ()

    lax.fori_loop(0, NCHUNK, body, (), unroll=False)

    plsc.subcore_barrier()
    pltpu.sync_copy(acc_sh.at[pl.ds(row0, RPT)], wout_v)
    pltpu.sync_copy(wout_v, out_hbm.at[cid, pl.ds(row0, RPT)])


_agg_call = pl.kernel(
    _agg_body,
    out_type=jax.ShapeDtypeStruct((NC, N_ACC, HID), jnp.float32),
    mesh=_sc_mesh(),
    compiler_params=pltpu.CompilerParams(use_tc_tiling_on_sc=False),
    scratch_types=[
        pltpu.VMEM((NCHUNK, CHUNK), jnp.int32),
        pltpu.VMEM((NCHUNK, CHUNK), jnp.int32),
        pltpu.VMEM((NBUF, CHUNK, HID), jnp.float32),
        pltpu.VMEM((RPT, HID), jnp.float32),
        pltpu.VMEM_SHARED((N_ACC, HID), jnp.float32),
        pltpu.SemaphoreType.DMA((NBUF,)),
    ],
)


def _tc1_body(x_ref, w_ref, c_ref, hs_ref, dinv_ref):
    s = c_ref[0, :, 0:1] + c_ref[1, :, 0:1]
    dinv = lax.rsqrt(1.0 + s)
    h = jnp.dot(x_ref[...], w_ref[...],
                preferred_element_type=jnp.float32)
    hs_ref[...] = dinv * h
    dinv_ref[...] = dinv


_tc1_call = pl.pallas_call(
    _tc1_body,
    grid=(N // BN,),
    in_specs=[
        pl.BlockSpec((BN, IN_CH), lambda i: (i, 0)),
        pl.BlockSpec((IN_CH, HID), lambda i: (0, 0)),
        pl.BlockSpec((NC, BN, 8), lambda i: (0, i, 0)),
    ],
    out_specs=[
        pl.BlockSpec((BN, HID), lambda i: (i, 0)),
        pl.BlockSpec((BN, 1), lambda i: (i, 0)),
    ],
    out_shape=[
        jax.ShapeDtypeStruct((N, HID), jnp.float32),
        jax.ShapeDtypeStruct((N, 1), jnp.float32),
    ],
)


def _tc2_body(p_ref, hs1_ref, dinv_ref, b_ref, w_ref, hs2_ref):
    dinv = dinv_ref[...]
    h = p_ref[0] + p_ref[1] + hs1_ref[...]
    h = jnp.maximum(dinv * h + b_ref[...], 0.0)
    hs2_ref[...] = dinv * jnp.dot(h, w_ref[...],
                                  preferred_element_type=jnp.float32)


_tc2_call = pl.pallas_call(
    _tc2_body,
    grid=(N // BN,),
    in_specs=[
        pl.BlockSpec((NC, BN, HID), lambda i: (0, i, 0)),
        pl.BlockSpec((BN, HID), lambda i: (i, 0)),
        pl.BlockSpec((BN, 1), lambda i: (i, 0)),
        pl.BlockSpec((1, HID), lambda i: (0, 0)),
        pl.BlockSpec((HID, HID), lambda i: (0, 0)),
    ],
    out_specs=pl.BlockSpec((BN, HID), lambda i: (i, 0)),
    out_shape=jax.ShapeDtypeStruct((N, HID), jnp.float32),
)


def _tc3_body(p_ref, hs2_ref, dinv_ref, b_ref, batch_ref, wc_ref, bc_ref,
              out_ref, seg_acc, cnt_acc):
    i = pl.program_id(0)

    @pl.when(i == 0)
    def _():
        seg_acc[...] = jnp.zeros_like(seg_acc)
        cnt_acc[...] = jnp.zeros_like(cnt_acc)

    dinv = dinv_ref[...]
    h = p_ref[0] + p_ref[1] + hs2_ref[...]
    h = jnp.maximum(dinv * h + b_ref[...], 0.0)

    gid = lax.broadcasted_iota(jnp.int32, (BN, G), 1)
    onehot = (batch_ref[...] == gid).astype(jnp.float32)
    seg_acc[...] += lax.dot_general(onehot, h, (((0,), (0,)), ((), ())),
                                    precision=_PREC,
                                    preferred_element_type=jnp.float32)
    cnt_acc[...] += lax.dot_general(onehot, jnp.ones((BN, 1), jnp.float32),
                                    (((0,), (0,)), ((), ())),
                                    precision=_PREC,
                                    preferred_element_type=jnp.float32)

    @pl.when(i == pl.num_programs(0) - 1)
    def _():
        mean = seg_acc[...] / jnp.maximum(cnt_acc[...], 1.0)
        out_ref[...] = jnp.dot(mean, wc_ref[...],
                               preferred_element_type=jnp.float32) + bc_ref[...]


_tc3_call = pl.pallas_call(
    _tc3_body,
    grid=(N // BN,),
    in_specs=[
        pl.BlockSpec((NC, BN, HID), lambda i: (0, i, 0)),
        pl.BlockSpec((BN, HID), lambda i: (i, 0)),
        pl.BlockSpec((BN, 1), lambda i: (i, 0)),
        pl.BlockSpec((1, HID), lambda i: (0, 0)),
        pl.BlockSpec((BN, 1), lambda i: (i, 0)),
        pl.BlockSpec((HID, OUT), lambda i: (0, 0)),
        pl.BlockSpec((1, OUT), lambda i: (0, 0)),
    ],
    out_specs=pl.BlockSpec((G, OUT), lambda i: (0, 0)),
    out_shape=jax.ShapeDtypeStruct((G, OUT), jnp.float32),
    scratch_shapes=[
        pltpu.VMEM((G, HID), jnp.float32),
        pltpu.VMEM((G, 1), jnp.float32),
    ],
)


def kernel(x, edge_index, batch, W1, b1, W2, b2, Wc, bc):
    src = edge_index[0].astype(jnp.int32)
    dst = edge_index[1].astype(jnp.int32)
    pad = E_PAD - E
    srcp = jnp.concatenate([src, jnp.zeros((pad,), jnp.int32)]).reshape(NW, NCHUNK, CHUNK)
    dstp = jnp.concatenate([dst, jnp.full((pad,), DUMMY, jnp.int32)]).reshape(NW, NCHUNK, CHUNK)

    ones8 = jnp.zeros((CHUNK, 8), jnp.float32).at[:, 0].set(1.0)
    zero8 = jnp.zeros((RPT, 8), jnp.float32)
    zero64 = jnp.zeros((CHUNK, HID), jnp.float32)

    c = _deg_call(dstp, ones8, zero8)
    hs1, dinv = _tc1_call(x, W1, c[:, :N, :])

    p1 = _agg_call(hs1, srcp, dstp, zero64)
    hs2 = _tc2_call(p1[:, :N, :], hs1, dinv, b1.reshape(1, HID), W2)

    p2 = _agg_call(hs2, srcp, dstp, zero64)
    out = _tc3_call(p2[:, :N, :], hs2, dinv, b2.reshape(1, HID),
                    batch.astype(jnp.int32).reshape(N, 1), Wc, bc.reshape(1, OUT))
    return out
